# baseline (speedup 1.0000x reference)
import functools

import jax
import jax.numpy as jnp
from jax import lax
from jax.experimental import pallas as pl
from jax.experimental.pallas import tpu as pltpu
from jax.experimental.pallas import tpu_sc as plsc

L = 16
NG = 8
FW = 4 * L
NEG_INF = float("-inf")


def _insert(state, v, vs=None):
    acc, m1, m2, m3 = state
    acc = acc + (v if vs is None else vs)
    hi1 = jnp.maximum(m1, v)
    lo1 = jnp.minimum(m1, v)
    hi2 = jnp.maximum(m2, lo1)
    lo2 = jnp.minimum(m2, lo1)
    hi3 = jnp.maximum(m3, lo2)
    return (acc, hi1, hi2, hi3)


def _merge_states(a, b):
    acc, m1, m2, m3 = a
    b_acc, b1, b2, b3 = b
    acc = acc + b_acc
    _, m1, m2, m3 = _insert((acc, m1, m2, m3), b1, vs=jnp.zeros_like(b1))
    hi2 = jnp.maximum(m2, b2)
    lo2 = jnp.minimum(m2, b2)
    m2, m3 = hi2, jnp.maximum(m3, lo2)
    m3 = jnp.maximum(m3, jnp.minimum(m2, b3))
    return acc, m1, m2, m3


def _chunk_reduce(buf, n_vec, states):
    def body(jj, sts):
        return tuple(
            _insert(sts[g], buf[jj, pl.ds(g * L, L)]) for g in range(NG)
        )

    return lax.fori_loop(0, n_vec, body, states, unroll=2)


def _make_tc_call(TC_COLS, R, BJ=512):
    assert TC_COLS % BJ == 0

    def body(x_ref, o_ref, acc, m1, m2, m3):
        i = pl.program_id(0)

        @pl.when(i == 0)
        def _init():
            acc[...] = jnp.zeros((8, R), jnp.float32)
            m1[...] = jnp.full((8, R), NEG_INF)
            m2[...] = jnp.full((8, R), NEG_INF)
            m3[...] = jnp.full((8, R), NEG_INF)

        def step(t, st):
            v = x_ref[pl.ds(t * 8, 8), :]
            return _insert(st, v)

        st = lax.fori_loop(
            0, BJ // 8, step, (acc[...], m1[...], m2[...], m3[...]),
            unroll=2,
        )
        acc[...], m1[...], m2[...], m3[...] = st

        @pl.when(i == pl.num_programs(0) - 1)
        def _fin():
            st8 = (acc[...], m1[...], m2[...], m3[...])
            cur = st8
            for width in (4, 2, 1):
                top = tuple(v[:width] for v in cur)
                bot = tuple(v[width : 2 * width] for v in cur)
                cur = _merge_states(top, bot)
            o_ref[...] = jnp.concatenate(cur, axis=0)

    return pl.pallas_call(
        body,
        grid=(TC_COLS // BJ,),
        in_specs=[pl.BlockSpec((BJ, R), lambda i: (i, 0))],
        out_specs=pl.BlockSpec((4, R), lambda i: (0, 0)),
        out_shape=jax.ShapeDtypeStruct((4, R), jnp.float32),
        scratch_shapes=[pltpu.VMEM((8, R), jnp.float32)] * 4,
    )


def _make_sc_call(N, R, SC_START):
    info = plsc.get_sparse_core_info()
    NC, NS = info.num_cores, info.num_subcores
    NW = NC * NS
    assert R == NG * L
    n_tiles = N // 8
    sc_tiles = n_tiles - SC_START // 8
    main_tiles = (sc_tiles // NW) * NW
    MAIN = SC_START + main_tiles * 8
    STRIPE = main_tiles // NW * 8
    tail_tiles = n_tiles - SC_START // 8 - main_tiles
    assert tail_tiles <= NW and (N - MAIN) == tail_tiles * 8
    NCH = 10
    assert STRIPE % (8 * NCH) == 0
    CJ = STRIPE // NCH
    assert CJ % 8 == 0 and CJ * NCH == STRIPE

    mesh = plsc.VectorSubcoreMesh(core_axis_name="c", subcore_axis_name="s")

    @functools.partial(
        pl.kernel,
        out_type=(
            jax.ShapeDtypeStruct((NC, NG, 4, L), jnp.float32),
            jax.ShapeDtypeStruct((NC, NG, NS, FW), jnp.float32),
        ),
        mesh=mesh,
        compiler_params=pltpu.CompilerParams(needs_layout_passes=False),
        scratch_types=[
            pltpu.VMEM((CJ, R), jnp.float32),
            pltpu.VMEM((CJ, R), jnp.float32),
            pltpu.VMEM((8, R), jnp.float32),
            pltpu.VMEM((NG, FW), jnp.float32),
            pltpu.VMEM((NS, FW), jnp.float32),
            pltpu.VMEM((4, L), jnp.float32),
            pltpu.SemaphoreType.DMA,
            pltpu.SemaphoreType.DMA,
            pltpu.SemaphoreType.DMA,
        ],
    )
    def sc_call(
        xt_hbm, out_hbm, stage_hbm, buf0, buf1, tailbuf, statebuf, gatherbuf,
        mergebuf, sem0, sem1, semt,
    ):
        c = lax.axis_index("c")
        s = lax.axis_index("s")
        w = c * NS + s
        j0 = SC_START + w * STRIPE
        bufs = (buf0, buf1)
        sems = (sem0, sem1)

        def copy(k):
            return pltpu.make_async_copy(
                xt_hbm.at[pl.ds(j0 + k * CJ, CJ)], bufs[k % 2], sems[k % 2]
            )

        tw = jnp.where(w < tail_tiles, w, w - tail_tiles)
        tail_copy = pltpu.make_async_copy(
            xt_hbm.at[pl.ds(MAIN + 8 * tw, 8)], tailbuf, semt
        )
        copy(0).start()
        tail_copy.start()

        zeros = jnp.zeros((L,), jnp.float32)
        ninf = jnp.full((L,), NEG_INF)
        states = tuple((zeros, ninf, ninf, ninf) for _ in range(NG))

        for k in range(NCH):
            if k + 1 < NCH:
                copy(k + 1).start()
            copy(k).wait()
            states = _chunk_reduce(bufs[k % 2], CJ, states)

        tail_copy.wait()
        valid = w < tail_tiles
        states = list(states)
        for jj in range(8):
            for g in range(NG):
                v = tailbuf[jj, pl.ds(g * L, L)]
                vt = jnp.where(valid, v, NEG_INF)
                vs = jnp.where(valid, v, 0.0)
                states[g] = _insert(states[g], vt, vs=vs)

        for g in range(NG):
            for f in range(4):
                statebuf[g, pl.ds(f * L, L)] = states[g][f]
            pltpu.sync_copy(statebuf.at[g], stage_hbm.at[c, g, s])

        plsc.subcore_barrier()

        @pl.when(s < NG)
        def _reduce():
            pltpu.sync_copy(stage_hbm.at[c, s], gatherbuf)

            def block(w2):
                return tuple(
                    gatherbuf[w2, pl.ds(f * L, L)] for f in range(4)
                )

            st = block(0)
            for w2 in range(1, NS):
                st = _merge_states(st, block(w2))
            mergebuf[0] = st[0]
            mergebuf[1] = st[1]
            mergebuf[2] = st[2]
            mergebuf[3] = st[3]
            pltpu.sync_copy(mergebuf, out_hbm.at[c, s])

    return sc_call


TC_COLS = 46080


def kernel(x, head_len):
    del head_len
    R, N = x.shape
    xt = x.T
    out, _ = _make_sc_call(N, R, TC_COLS)(xt)
    tc = _make_tc_call(TC_COLS, R)(xt)
    a = (out[0, :, 0], out[0, :, 1], out[0, :, 2], out[0, :, 3])
    b = (out[1, :, 0], out[1, :, 1], out[1, :, 2], out[1, :, 3])
    acc, m1, m2, m3 = _merge_states(a, b)
    t = tuple(tc[f].reshape(NG, L) for f in range(4))
    acc, _m1, m2, m3 = _merge_states((acc, m1, m2, m3), t)
    return jnp.sum(acc - m2 - 2.0 * m3)


# speedup vs baseline: 1.0844x; 1.0844x over previous
import functools

import jax
import jax.numpy as jnp
from jax import lax
from jax.experimental import pallas as pl
from jax.experimental.pallas import tpu as pltpu
from jax.experimental.pallas import tpu_sc as plsc

L = 16
NG = 8
FW = 4 * L
NEG_INF = float("-inf")


def _insert(state, v, vs=None):
    acc, m1, m2, m3 = state
    acc = acc + (v if vs is None else vs)
    hi1 = jnp.maximum(m1, v)
    lo1 = jnp.minimum(m1, v)
    hi2 = jnp.maximum(m2, lo1)
    lo2 = jnp.minimum(m2, lo1)
    hi3 = jnp.maximum(m3, lo2)
    return (acc, hi1, hi2, hi3)


def _merge_states(a, b):
    acc, m1, m2, m3 = a
    b_acc, b1, b2, b3 = b
    acc = acc + b_acc
    _, m1, m2, m3 = _insert((acc, m1, m2, m3), b1, vs=jnp.zeros_like(b1))
    hi2 = jnp.maximum(m2, b2)
    lo2 = jnp.minimum(m2, b2)
    m2, m3 = hi2, jnp.maximum(m3, lo2)
    m3 = jnp.maximum(m3, jnp.minimum(m2, b3))
    return acc, m1, m2, m3


def _chunk_reduce(buf, n_vec, states):
    def body(jj, sts):
        return tuple(
            _insert(sts[g], buf[jj, pl.ds(g * L, L)]) for g in range(NG)
        )

    return lax.fori_loop(0, n_vec, body, states, unroll=2)


def _make_tc_call(TC_COLS, R, BJ=512):
    assert TC_COLS % BJ == 0

    KC = 4

    def body(x_ref, o_ref, acc, m1, m2, m3):
        i = pl.program_id(0)

        @pl.when(i == 0)
        def _init():
            acc[...] = jnp.zeros((KC * 8, R), jnp.float32)
            m1[...] = jnp.full((KC * 8, R), NEG_INF)
            m2[...] = jnp.full((KC * 8, R), NEG_INF)
            m3[...] = jnp.full((KC * 8, R), NEG_INF)

        states = [
            tuple(f[k * 8 : (k + 1) * 8, :] for f in (acc[...], m1[...], m2[...], m3[...]))
            for k in range(KC)
        ]
        for t in range(BJ // 8):
            v = x_ref[t * 8 : (t + 1) * 8, :]
            states[t % KC] = _insert(states[t % KC], v)
        acc[...] = jnp.concatenate([st[0] for st in states], axis=0)
        m1[...] = jnp.concatenate([st[1] for st in states], axis=0)
        m2[...] = jnp.concatenate([st[2] for st in states], axis=0)
        m3[...] = jnp.concatenate([st[3] for st in states], axis=0)

        @pl.when(i == pl.num_programs(0) - 1)
        def _fin():
            cur = (acc[...], m1[...], m2[...], m3[...])
            width = KC * 8 // 2
            while width >= 1:
                top = tuple(v[:width] for v in cur)
                bot = tuple(v[width : 2 * width] for v in cur)
                cur = _merge_states(top, bot)
                width //= 2
            o_ref[...] = jnp.concatenate(cur, axis=0)

    return pl.pallas_call(
        body,
        grid=(TC_COLS // BJ,),
        in_specs=[pl.BlockSpec((BJ, R), lambda i: (i, 0))],
        out_specs=pl.BlockSpec((4, R), lambda i: (0, 0)),
        out_shape=jax.ShapeDtypeStruct((4, R), jnp.float32),
        scratch_shapes=[pltpu.VMEM((KC * 8, R), jnp.float32)] * 4,
    )


def _make_sc_call(N, R, SC_START):
    info = plsc.get_sparse_core_info()
    NC, NS = info.num_cores, info.num_subcores
    NW = NC * NS
    assert R == NG * L
    n_tiles = N // 8
    sc_tiles = n_tiles - SC_START // 8
    main_tiles = (sc_tiles // NW) * NW
    MAIN = SC_START + main_tiles * 8
    STRIPE = main_tiles // NW * 8
    tail_tiles = n_tiles - SC_START // 8 - main_tiles
    assert tail_tiles <= NW and (N - MAIN) == tail_tiles * 8
    NCH = 10
    assert STRIPE % (8 * NCH) == 0
    CJ = STRIPE // NCH
    assert CJ % 8 == 0 and CJ * NCH == STRIPE

    mesh = plsc.VectorSubcoreMesh(core_axis_name="c", subcore_axis_name="s")

    @functools.partial(
        pl.kernel,
        out_type=(
            jax.ShapeDtypeStruct((NC, NG, 4, L), jnp.float32),
            jax.ShapeDtypeStruct((NC, NG, NS, FW), jnp.float32),
        ),
        mesh=mesh,
        compiler_params=pltpu.CompilerParams(needs_layout_passes=False),
        scratch_types=[
            pltpu.VMEM((CJ, R), jnp.float32),
            pltpu.VMEM((CJ, R), jnp.float32),
            pltpu.VMEM((8, R), jnp.float32),
            pltpu.VMEM((NG, FW), jnp.float32),
            pltpu.VMEM((NS, FW), jnp.float32),
            pltpu.VMEM((4, L), jnp.float32),
            pltpu.SemaphoreType.DMA,
            pltpu.SemaphoreType.DMA,
            pltpu.SemaphoreType.DMA,
        ],
    )
    def sc_call(
        xt_hbm, out_hbm, stage_hbm, buf0, buf1, tailbuf, statebuf, gatherbuf,
        mergebuf, sem0, sem1, semt,
    ):
        c = lax.axis_index("c")
        s = lax.axis_index("s")
        w = c * NS + s
        j0 = SC_START + w * STRIPE
        bufs = (buf0, buf1)
        sems = (sem0, sem1)

        def copy(k):
            return pltpu.make_async_copy(
                xt_hbm.at[pl.ds(j0 + k * CJ, CJ)], bufs[k % 2], sems[k % 2]
            )

        tw = jnp.where(w < tail_tiles, w, w - tail_tiles)
        tail_copy = pltpu.make_async_copy(
            xt_hbm.at[pl.ds(MAIN + 8 * tw, 8)], tailbuf, semt
        )
        copy(0).start()
        tail_copy.start()

        zeros = jnp.zeros((L,), jnp.float32)
        ninf = jnp.full((L,), NEG_INF)
        states = tuple((zeros, ninf, ninf, ninf) for _ in range(NG))

        for k in range(NCH):
            if k + 1 < NCH:
                copy(k + 1).start()
            copy(k).wait()
            states = _chunk_reduce(bufs[k % 2], CJ, states)

        tail_copy.wait()
        valid = w < tail_tiles
        states = list(states)
        for jj in range(8):
            for g in range(NG):
                v = tailbuf[jj, pl.ds(g * L, L)]
                vt = jnp.where(valid, v, NEG_INF)
                vs = jnp.where(valid, v, 0.0)
                states[g] = _insert(states[g], vt, vs=vs)

        for g in range(NG):
            for f in range(4):
                statebuf[g, pl.ds(f * L, L)] = states[g][f]
            pltpu.sync_copy(statebuf.at[g], stage_hbm.at[c, g, s])

        plsc.subcore_barrier()

        @pl.when(s < NG)
        def _reduce():
            pltpu.sync_copy(stage_hbm.at[c, s], gatherbuf)

            def block(w2):
                return tuple(
                    gatherbuf[w2, pl.ds(f * L, L)] for f in range(4)
                )

            st = block(0)
            for w2 in range(1, NS):
                st = _merge_states(st, block(w2))
            mergebuf[0] = st[0]
            mergebuf[1] = st[1]
            mergebuf[2] = st[2]
            mergebuf[3] = st[3]
            pltpu.sync_copy(mergebuf, out_hbm.at[c, s])

    return sc_call


TC_COLS = 46080


def kernel(x, head_len):
    del head_len
    R, N = x.shape
    xt = x.T
    out, _ = _make_sc_call(N, R, TC_COLS)(xt)
    tc = _make_tc_call(TC_COLS, R)(xt)
    a = (out[0, :, 0], out[0, :, 1], out[0, :, 2], out[0, :, 3])
    b = (out[1, :, 0], out[1, :, 1], out[1, :, 2], out[1, :, 3])
    acc, m1, m2, m3 = _merge_states(a, b)
    t = tuple(tc[f].reshape(NG, L) for f in range(4))
    acc, _m1, m2, m3 = _merge_states((acc, m1, m2, m3), t)
    return jnp.sum(acc - m2 - 2.0 * m3)


# speedup vs baseline: 1.0963x; 1.0109x over previous
import functools

import jax
import jax.numpy as jnp
from jax import lax
from jax.experimental import pallas as pl
from jax.experimental.pallas import tpu as pltpu
from jax.experimental.pallas import tpu_sc as plsc

L = 16
NG = 8
FW = 4 * L
NEG_INF = float("-inf")


def _insert(state, v, vs=None):
    acc, m1, m2, m3 = state
    acc = acc + (v if vs is None else vs)
    hi1 = jnp.maximum(m1, v)
    lo1 = jnp.minimum(m1, v)
    hi2 = jnp.maximum(m2, lo1)
    lo2 = jnp.minimum(m2, lo1)
    hi3 = jnp.maximum(m3, lo2)
    return (acc, hi1, hi2, hi3)


def _merge_states(a, b):
    acc, m1, m2, m3 = a
    b_acc, b1, b2, b3 = b
    acc = acc + b_acc
    _, m1, m2, m3 = _insert((acc, m1, m2, m3), b1, vs=jnp.zeros_like(b1))
    hi2 = jnp.maximum(m2, b2)
    lo2 = jnp.minimum(m2, b2)
    m2, m3 = hi2, jnp.maximum(m3, lo2)
    m3 = jnp.maximum(m3, jnp.minimum(m2, b3))
    return acc, m1, m2, m3


def _chunk_reduce(buf, n_vec, states):
    def body(jj, sts):
        return tuple(
            _insert(sts[g], buf[jj, pl.ds(g * L, L)]) for g in range(NG)
        )

    return lax.fori_loop(0, n_vec, body, states, unroll=2)


def _make_tc_call(TC_COLS, R, BJ=512):
    assert TC_COLS % BJ == 0

    KC = 4

    def body(x_ref, o_ref, *scr):
        i = pl.program_id(0)
        refs = [scr[k * 4 : (k + 1) * 4] for k in range(KC)]

        @pl.when(i == 0)
        def _init():
            for k in range(KC):
                refs[k][0][...] = jnp.zeros((8, R), jnp.float32)
                for f in range(1, 4):
                    refs[k][f][...] = jnp.full((8, R), NEG_INF)

        states = [tuple(r[...] for r in refs[k]) for k in range(KC)]
        for t in range(BJ // 8):
            v = x_ref[t * 8 : (t + 1) * 8, :]
            states[t % KC] = _insert(states[t % KC], v)
        for k in range(KC):
            for f in range(4):
                refs[k][f][...] = states[k][f]

        @pl.when(i == pl.num_programs(0) - 1)
        def _fin():
            cur = states[0]
            for k in range(1, KC):
                cur = _merge_states(cur, states[k])
            for width in (4, 2, 1):
                top = tuple(v[:width] for v in cur)
                bot = tuple(v[width : 2 * width] for v in cur)
                cur = _merge_states(top, bot)
            o_ref[...] = jnp.concatenate(cur, axis=0)

    return pl.pallas_call(
        body,
        grid=(TC_COLS // BJ,),
        in_specs=[pl.BlockSpec((BJ, R), lambda i: (i, 0))],
        out_specs=pl.BlockSpec((4, R), lambda i: (0, 0)),
        out_shape=jax.ShapeDtypeStruct((4, R), jnp.float32),
        scratch_shapes=[pltpu.VMEM((8, R), jnp.float32)] * (4 * KC),
    )


def _make_sc_call(N, R, SC_START):
    info = plsc.get_sparse_core_info()
    NC, NS = info.num_cores, info.num_subcores
    NW = NC * NS
    assert R == NG * L
    n_tiles = N // 8
    sc_tiles = n_tiles - SC_START // 8
    main_tiles = (sc_tiles // NW) * NW
    MAIN = SC_START + main_tiles * 8
    STRIPE = main_tiles // NW * 8
    tail_tiles = n_tiles - SC_START // 8 - main_tiles
    assert tail_tiles <= NW and (N - MAIN) == tail_tiles * 8
    NCH = 10
    assert STRIPE % (8 * NCH) == 0
    CJ = STRIPE // NCH
    assert CJ % 8 == 0 and CJ * NCH == STRIPE

    mesh = plsc.VectorSubcoreMesh(core_axis_name="c", subcore_axis_name="s")

    @functools.partial(
        pl.kernel,
        out_type=(
            jax.ShapeDtypeStruct((NC, NG, 4, L), jnp.float32),
            jax.ShapeDtypeStruct((NC, NG, NS, FW), jnp.float32),
        ),
        mesh=mesh,
        compiler_params=pltpu.CompilerParams(needs_layout_passes=False),
        scratch_types=[
            pltpu.VMEM((CJ, R), jnp.float32),
            pltpu.VMEM((CJ, R), jnp.float32),
            pltpu.VMEM((8, R), jnp.float32),
            pltpu.VMEM((NG, FW), jnp.float32),
            pltpu.VMEM((NS, FW), jnp.float32),
            pltpu.VMEM((4, L), jnp.float32),
            pltpu.SemaphoreType.DMA,
            pltpu.SemaphoreType.DMA,
            pltpu.SemaphoreType.DMA,
        ],
    )
    def sc_call(
        xt_hbm, out_hbm, stage_hbm, buf0, buf1, tailbuf, statebuf, gatherbuf,
        mergebuf, sem0, sem1, semt,
    ):
        c = lax.axis_index("c")
        s = lax.axis_index("s")
        w = c * NS + s
        j0 = SC_START + w * STRIPE
        bufs = (buf0, buf1)
        sems = (sem0, sem1)

        def copy(k):
            return pltpu.make_async_copy(
                xt_hbm.at[pl.ds(j0 + k * CJ, CJ)], bufs[k % 2], sems[k % 2]
            )

        tw = jnp.where(w < tail_tiles, w, w - tail_tiles)
        tail_copy = pltpu.make_async_copy(
            xt_hbm.at[pl.ds(MAIN + 8 * tw, 8)], tailbuf, semt
        )
        copy(0).start()
        tail_copy.start()

        zeros = jnp.zeros((L,), jnp.float32)
        ninf = jnp.full((L,), NEG_INF)
        states = tuple((zeros, ninf, ninf, ninf) for _ in range(NG))

        for k in range(NCH):
            if k + 1 < NCH:
                copy(k + 1).start()
            copy(k).wait()
            states = _chunk_reduce(bufs[k % 2], CJ, states)

        tail_copy.wait()
        valid = w < tail_tiles
        states = list(states)
        for jj in range(8):
            for g in range(NG):
                v = tailbuf[jj, pl.ds(g * L, L)]
                vt = jnp.where(valid, v, NEG_INF)
                vs = jnp.where(valid, v, 0.0)
                states[g] = _insert(states[g], vt, vs=vs)

        for g in range(NG):
            for f in range(4):
                statebuf[g, pl.ds(f * L, L)] = states[g][f]
            pltpu.sync_copy(statebuf.at[g], stage_hbm.at[c, g, s])

        plsc.subcore_barrier()

        @pl.when(s < NG)
        def _reduce():
            pltpu.sync_copy(stage_hbm.at[c, s], gatherbuf)

            def block(w2):
                return tuple(
                    gatherbuf[w2, pl.ds(f * L, L)] for f in range(4)
                )

            st = block(0)
            for w2 in range(1, NS):
                st = _merge_states(st, block(w2))
            mergebuf[0] = st[0]
            mergebuf[1] = st[1]
            mergebuf[2] = st[2]
            mergebuf[3] = st[3]
            pltpu.sync_copy(mergebuf, out_hbm.at[c, s])

    return sc_call


TC_COLS = 46080


def kernel(x, head_len):
    del head_len
    R, N = x.shape
    xt = x.T
    out, _ = _make_sc_call(N, R, TC_COLS)(xt)
    tc = _make_tc_call(TC_COLS, R)(xt)
    a = (out[0, :, 0], out[0, :, 1], out[0, :, 2], out[0, :, 3])
    b = (out[1, :, 0], out[1, :, 1], out[1, :, 2], out[1, :, 3])
    acc, m1, m2, m3 = _merge_states(a, b)
    t = tuple(tc[f].reshape(NG, L) for f in range(4))
    acc, _m1, m2, m3 = _merge_states((acc, m1, m2, m3), t)
    return jnp.sum(acc - m2 - 2.0 * m3)


# speedup vs baseline: 1.2952x; 1.1814x over previous
import functools

import jax
import jax.numpy as jnp
from jax import lax
from jax.experimental import pallas as pl
from jax.experimental.pallas import tpu as pltpu
from jax.experimental.pallas import tpu_sc as plsc

L = 16
NG = 8
FW = 4 * L
NEG_INF = float("-inf")


def _insert(state, v, vs=None):
    acc, m1, m2, m3 = state
    acc = acc + (v if vs is None else vs)
    hi1 = jnp.maximum(m1, v)
    lo1 = jnp.minimum(m1, v)
    hi2 = jnp.maximum(m2, lo1)
    lo2 = jnp.minimum(m2, lo1)
    hi3 = jnp.maximum(m3, lo2)
    return (acc, hi1, hi2, hi3)


def _merge_states(a, b):
    acc, m1, m2, m3 = a
    b_acc, b1, b2, b3 = b
    acc = acc + b_acc
    _, m1, m2, m3 = _insert((acc, m1, m2, m3), b1, vs=jnp.zeros_like(b1))
    hi2 = jnp.maximum(m2, b2)
    lo2 = jnp.minimum(m2, b2)
    m2, m3 = hi2, jnp.maximum(m3, lo2)
    m3 = jnp.maximum(m3, jnp.minimum(m2, b3))
    return acc, m1, m2, m3


def _chunk_reduce(buf, n_vec, states):
    def body(jj, sts):
        return tuple(
            _insert(sts[g], buf[jj, pl.ds(g * L, L)]) for g in range(NG)
        )

    return lax.fori_loop(0, n_vec, body, states, unroll=4)


def _make_sc_call(N, R):
    info = plsc.get_sparse_core_info()
    NC, NS = info.num_cores, info.num_subcores
    NW = NC * NS
    assert R == NG * L
    n_tiles = N // 8
    main_tiles = n_tiles // NW * NW
    MAIN = main_tiles * 8
    STRIPE = MAIN // NW
    tail_tiles = n_tiles - main_tiles
    assert tail_tiles <= NW and (N - MAIN) == tail_tiles * 8
    NCH = 10
    CJ = STRIPE // NCH
    assert CJ % 8 == 0 and CJ * NCH == STRIPE

    mesh = plsc.VectorSubcoreMesh(core_axis_name="c", subcore_axis_name="s")

    @functools.partial(
        pl.kernel,
        out_type=(
            jax.ShapeDtypeStruct((NC, NG, 4, L), jnp.float32),
            jax.ShapeDtypeStruct((NC, NG, NS, FW), jnp.float32),
        ),
        mesh=mesh,
        compiler_params=pltpu.CompilerParams(needs_layout_passes=False),
        scratch_types=[
            pltpu.VMEM((CJ, R), jnp.float32),
            pltpu.VMEM((CJ, R), jnp.float32),
            pltpu.VMEM((8, R), jnp.float32),
            pltpu.VMEM((NG, FW), jnp.float32),
            pltpu.VMEM((NS, FW), jnp.float32),
            pltpu.VMEM((4, L), jnp.float32),
            pltpu.SemaphoreType.DMA,
            pltpu.SemaphoreType.DMA,
            pltpu.SemaphoreType.DMA,
        ],
    )
    def sc_call(
        xt_hbm, out_hbm, stage_hbm, buf0, buf1, tailbuf, statebuf, gatherbuf,
        mergebuf, sem0, sem1, semt,
    ):
        c = lax.axis_index("c")
        s = lax.axis_index("s")
        w = c * NS + s
        j0 = w * STRIPE
        bufs = (buf0, buf1)
        sems = (sem0, sem1)

        def copy(k):
            return pltpu.make_async_copy(
                xt_hbm.at[pl.ds(j0 + k * CJ, CJ)], bufs[k % 2], sems[k % 2]
            )

        tw = jnp.where(w < tail_tiles, w, w - tail_tiles)
        tail_copy = pltpu.make_async_copy(
            xt_hbm.at[pl.ds(MAIN + 8 * tw, 8)], tailbuf, semt
        )
        copy(0).start()
        tail_copy.start()

        zeros = jnp.zeros((L,), jnp.float32)
        ninf = jnp.full((L,), NEG_INF)
        states = tuple((zeros, ninf, ninf, ninf) for _ in range(NG))

        for k in range(NCH):
            if k + 1 < NCH:
                copy(k + 1).start()
            copy(k).wait()
            states = _chunk_reduce(bufs[k % 2], CJ, states)

        tail_copy.wait()
        valid = w < tail_tiles
        states = list(states)
        for jj in range(8):
            for g in range(NG):
                v = tailbuf[jj, pl.ds(g * L, L)]
                vt = jnp.where(valid, v, NEG_INF)
                vs = jnp.where(valid, v, 0.0)
                states[g] = _insert(states[g], vt, vs=vs)

        for g in range(NG):
            for f in range(4):
                statebuf[g, pl.ds(f * L, L)] = states[g][f]
            pltpu.sync_copy(statebuf.at[g], stage_hbm.at[c, g, s])

        plsc.subcore_barrier()

        @pl.when(s < NG)
        def _reduce():
            pltpu.sync_copy(stage_hbm.at[c, s], gatherbuf)

            def block(w2):
                return tuple(
                    gatherbuf[w2, pl.ds(f * L, L)] for f in range(4)
                )

            st = block(0)
            for w2 in range(1, NS):
                st = _merge_states(st, block(w2))
            mergebuf[0] = st[0]
            mergebuf[1] = st[1]
            mergebuf[2] = st[2]
            mergebuf[3] = st[3]
            pltpu.sync_copy(mergebuf, out_hbm.at[c, s])

    return sc_call


def kernel(x, head_len):
    del head_len
    R, N = x.shape
    out, _ = _make_sc_call(N, R)(x.T)
    a = (out[0, :, 0], out[0, :, 1], out[0, :, 2], out[0, :, 3])
    b = (out[1, :, 0], out[1, :, 1], out[1, :, 2], out[1, :, 3])
    acc, _m1, m2, m3 = _merge_states(a, b)
    return jnp.sum(acc - m2 - 2.0 * m3)


# speedup vs baseline: 1.3434x; 1.0372x over previous
import functools

import jax
import jax.numpy as jnp
from jax import lax
from jax.experimental import pallas as pl
from jax.experimental.pallas import tpu as pltpu
from jax.experimental.pallas import tpu_sc as plsc

L = 16
NG = 8
FW = 4 * L
NEG_INF = float("-inf")


def _insert(state, v, vs=None):
    acc, m1, m2, m3 = state
    acc = acc + (v if vs is None else vs)
    hi1 = jnp.maximum(m1, v)
    lo1 = jnp.minimum(m1, v)
    hi2 = jnp.maximum(m2, lo1)
    lo2 = jnp.minimum(m2, lo1)
    hi3 = jnp.maximum(m3, lo2)
    return (acc, hi1, hi2, hi3)


def _merge_states(a, b):
    acc, m1, m2, m3 = a
    b_acc, b1, b2, b3 = b
    acc = acc + b_acc
    _, m1, m2, m3 = _insert((acc, m1, m2, m3), b1, vs=jnp.zeros_like(b1))
    hi2 = jnp.maximum(m2, b2)
    lo2 = jnp.minimum(m2, b2)
    m2, m3 = hi2, jnp.maximum(m3, lo2)
    m3 = jnp.maximum(m3, jnp.minimum(m2, b3))
    return acc, m1, m2, m3


def _chunk_reduce(buf, n_vec, states):
    def body(jj, sts):
        return tuple(
            _insert(sts[g], buf[jj, pl.ds(g * L, L)]) for g in range(NG)
        )

    return lax.fori_loop(0, n_vec, body, states, unroll=2)


def _make_sc_call(N, R):
    info = plsc.get_sparse_core_info()
    NC, NS = info.num_cores, info.num_subcores
    NW = NC * NS
    assert R == NG * L
    n_tiles = N // 8
    main_tiles = n_tiles // NW * NW
    MAIN = main_tiles * 8
    STRIPE = MAIN // NW
    tail_tiles = n_tiles - main_tiles
    assert tail_tiles <= NW and (N - MAIN) == tail_tiles * 8
    NCH = 10
    CJ = STRIPE // NCH
    assert CJ % 8 == 0 and CJ * NCH == STRIPE

    mesh = plsc.VectorSubcoreMesh(core_axis_name="c", subcore_axis_name="s")

    @functools.partial(
        pl.kernel,
        out_type=(
            jax.ShapeDtypeStruct((NC, NG, 4, L), jnp.float32),
            jax.ShapeDtypeStruct((NC, NG, NS, FW), jnp.float32),
        ),
        mesh=mesh,
        compiler_params=pltpu.CompilerParams(needs_layout_passes=False),
        scratch_types=[
            pltpu.VMEM((CJ, R), jnp.float32),
            pltpu.VMEM((CJ, R), jnp.float32),
            pltpu.VMEM((8, R), jnp.float32),
            pltpu.VMEM((NG, FW), jnp.float32),
            pltpu.VMEM((NS, FW), jnp.float32),
            pltpu.VMEM((4, L), jnp.float32),
            pltpu.SemaphoreType.DMA,
            pltpu.SemaphoreType.DMA,
            pltpu.SemaphoreType.DMA,
        ],
    )
    def sc_call(
        xt_hbm, out_hbm, stage_hbm, buf0, buf1, tailbuf, statebuf, gatherbuf,
        mergebuf, sem0, sem1, semt,
    ):
        c = lax.axis_index("c")
        s = lax.axis_index("s")
        w = c * NS + s
        j0 = w * STRIPE
        bufs = (buf0, buf1)
        sems = (sem0, sem1)

        def copy(k):
            return pltpu.make_async_copy(
                xt_hbm.at[pl.ds(j0 + k * CJ, CJ)], bufs[k % 2], sems[k % 2]
            )

        tw = jnp.where(w < tail_tiles, w, w - tail_tiles)
        tail_copy = pltpu.make_async_copy(
            xt_hbm.at[pl.ds(MAIN + 8 * tw, 8)], tailbuf, semt
        )
        copy(0).start()
        tail_copy.start()

        zeros = jnp.zeros((L,), jnp.float32)
        ninf = jnp.full((L,), NEG_INF)
        states = tuple((zeros, ninf, ninf, ninf) for _ in range(NG))

        for k in range(NCH):
            if k + 1 < NCH:
                copy(k + 1).start()
            copy(k).wait()
            states = _chunk_reduce(bufs[k % 2], CJ, states)

        tail_copy.wait()
        valid = w < tail_tiles
        states = list(states)
        for jj in range(8):
            for g in range(NG):
                v = tailbuf[jj, pl.ds(g * L, L)]
                vt = jnp.where(valid, v, NEG_INF)
                vs = jnp.where(valid, v, 0.0)
                states[g] = _insert(states[g], vt, vs=vs)

        for g in range(NG):
            for f in range(4):
                statebuf[g, pl.ds(f * L, L)] = states[g][f]
            pltpu.sync_copy(statebuf.at[g], stage_hbm.at[c, g, s])

        plsc.subcore_barrier()

        @pl.when(s < NG)
        def _reduce():
            pltpu.sync_copy(stage_hbm.at[c, s], gatherbuf)

            def block(w2):
                return tuple(
                    gatherbuf[w2, pl.ds(f * L, L)] for f in range(4)
                )

            st = block(0)
            for w2 in range(1, NS):
                st = _merge_states(st, block(w2))
            mergebuf[0] = st[0]
            mergebuf[1] = st[1]
            mergebuf[2] = st[2]
            mergebuf[3] = st[3]
            pltpu.sync_copy(mergebuf, out_hbm.at[c, s])

    return sc_call


def kernel(x, head_len):
    del head_len
    R, N = x.shape
    out, _ = _make_sc_call(N, R)(x.T)
    a = (out[0, :, 0], out[0, :, 1], out[0, :, 2], out[0, :, 3])
    b = (out[1, :, 0], out[1, :, 1], out[1, :, 2], out[1, :, 3])
    acc, _m1, m2, m3 = _merge_states(a, b)
    return jnp.sum(acc - m2 - 2.0 * m3)
